# baseline (speedup 1.0000x reference)
import functools

import jax
import jax.numpy as jnp
from jax import lax
from jax.experimental import pallas as pl
from jax.experimental.pallas import tpu as pltpu
from jax.experimental.pallas import tpu_sc as plsc

D_MODEL = 64
SEQ = 50
BATCH = 4096
NUM_CORES = 2
NUM_SUBCORES = 16
NW = NUM_CORES * NUM_SUBCORES
B_PER_W = BATCH // NW
TCH = 2
NCHUNK = SEQ // TCH
CROWS = TCH * B_PER_W
LANES = 16
BGROUPS = B_PER_W // LANES
PESLAB = TCH * D_MODEL * LANES

_mesh = plsc.VectorSubcoreMesh(core_axis_name="c", subcore_axis_name="s")


@functools.partial(
    pl.kernel,
    out_type=jax.ShapeDtypeStruct((SEQ, D_MODEL, BATCH), jnp.float32),
    mesh=_mesh,
    scratch_types=[
        pltpu.VMEM((B_PER_W * SEQ,), jnp.int32),
        pltpu.VMEM((CROWS, D_MODEL), jnp.float32),
        pltpu.VMEM((TCH, D_MODEL, B_PER_W), jnp.float32),
        pltpu.VMEM((PESLAB,), jnp.float32),
        pltpu.SemaphoreType.DMA,
    ],
    compiler_params=pltpu.CompilerParams(needs_layout_passes=False),
)
def _stay_embedding(codes_hbm, table_hbm, pe_hbm, out_hbm,
                    cod_v, rbuf, tbuf, pe_v, sem):
    wid = lax.axis_index("s") * NUM_CORES + lax.axis_index("c")
    b0 = wid * B_PER_W
    pltpu.sync_copy(codes_hbm.at[pl.ds(b0 * SEQ, B_PER_W * SEQ)], cod_v)
    lane_iota = lax.iota(jnp.int32, LANES)
    row_iota = lane_iota * TCH

    def chunk_body(ci, carry):
        t0 = ci * TCH
        pltpu.sync_copy(pe_hbm.at[pl.ds(ci * PESLAB, PESLAB)], pe_v)

        def fire_group(g, c2):
            bg = g // TCH
            tt = g - bg * TCH
            cidx = lane_iota * SEQ + (bg * LANES * SEQ + t0 + tt)
            vec = plsc.load_gather(cod_v, [cidx])
            slot = bg * LANES * TCH + tt
            for j in range(LANES):
                code = vec[j]
                pltpu.make_async_copy(
                    table_hbm.at[code], rbuf.at[slot + j * TCH], sem
                ).start()
            return c2

        lax.fori_loop(0, BGROUPS * TCH, fire_group, 0)

        def drain_row(j, c2):
            pltpu.make_async_copy(table_hbm.at[0], rbuf.at[0], sem).wait()
            return c2

        lax.fori_loop(0, CROWS, drain_row, 0)

        def trans_body(i, c2):
            tt = i // D_MODEL
            d = i - tt * D_MODEL
            pe_vec = pe_v[pl.ds(tt * (D_MODEL * LANES) + d * LANES, LANES)]
            dcol = jnp.broadcast_to(d, (LANES,))
            for bg in range(BGROUPS):
                ridx = row_iota + (bg * LANES * TCH + tt)
                vals = plsc.load_gather(rbuf, [ridx, dcol])
                tbuf[tt, d, pl.ds(bg * LANES, LANES)] = vals + pe_vec
            return c2

        lax.fori_loop(0, TCH * D_MODEL, trans_body, 0)
        pltpu.sync_copy(
            tbuf, out_hbm.at[pl.ds(t0, TCH), :, pl.ds(b0, B_PER_W)]
        )
        return carry

    lax.fori_loop(0, NCHUNK, chunk_body, 0)


def kernel(codes, table, pe):
    codes_flat = codes.reshape(BATCH * SEQ)
    pe_b = jnp.broadcast_to(
        pe[:SEQ, :, None], (SEQ, D_MODEL, LANES)
    ).reshape(SEQ * D_MODEL * LANES)
    out_t = _stay_embedding(codes_flat, table, pe_b)
    return out_t.transpose(2, 0, 1)
